# initial kernel scaffold (unmeasured)
import jax
import jax.numpy as jnp
from jax import lax
from jax.experimental import pallas as pl
from jax.experimental.pallas import tpu as pltpu


def kernel(
    x,
):
    def body(*refs):
        pass

    out_shape = jax.ShapeDtypeStruct(..., jnp.float32)
    return pl.pallas_call(body, out_shape=out_shape)(...)


# baseline (device time: 52908 ns/iter reference)
import jax
import jax.numpy as jnp
from jax import lax
from jax.experimental import pallas as pl
from jax.experimental.pallas import tpu as pltpu


def kernel(x):
    m, n = x.shape

    def body(x_ref, out_ref, comm_ref, send_sem, recv_sem):
        my_x = lax.axis_index("x")
        my_y = lax.axis_index("y")
        my_z = lax.axis_index("z")
        nbr = (my_x, 1 - my_y, my_z)

        barrier_sem = pltpu.get_barrier_semaphore()
        pl.semaphore_signal(
            barrier_sem, inc=1, device_id=nbr,
            device_id_type=pl.DeviceIdType.MESH,
        )
        pl.semaphore_wait(barrier_sem, 1)

        rdma = pltpu.make_async_remote_copy(
            src_ref=x_ref,
            dst_ref=comm_ref,
            send_sem=send_sem,
            recv_sem=recv_sem,
            device_id=nbr,
            device_id_type=pl.DeviceIdType.MESH,
        )
        rdma.start()
        rdma.wait()
        out_ref[...] = x_ref[...] + comm_ref[...]

    return pl.pallas_call(
        body,
        out_shape=jax.ShapeDtypeStruct((m, n), x.dtype),
        in_specs=[pl.BlockSpec(memory_space=pltpu.VMEM)],
        out_specs=pl.BlockSpec(memory_space=pltpu.VMEM),
        scratch_shapes=[
            pltpu.VMEM((m, n), x.dtype),
            pltpu.SemaphoreType.DMA,
            pltpu.SemaphoreType.DMA,
        ],
        compiler_params=pltpu.CompilerParams(collective_id=0),
    )(x)


# device time: 35633 ns/iter; 1.4848x vs baseline; 1.4848x over previous
import jax
import jax.numpy as jnp
from jax import lax
from jax.experimental import pallas as pl
from jax.experimental.pallas import tpu as pltpu

C = 8


def kernel(x):
    m, n = x.shape
    half = m // 2
    h = half // C

    def body(x_ref, out_ref, p1_buf, p1_send, p1_recv, p2_send, p2_recv):
        my_x = lax.axis_index("x")
        my_y = lax.axis_index("y")
        my_z = lax.axis_index("z")
        y_nbr = (my_x, 1 - my_y, my_z)
        x_nbr = (1 - my_x, my_y, my_z)

        my_off = my_x * half
        other_off = (1 - my_x) * half

        barrier_sem = pltpu.get_barrier_semaphore()
        for nbr in (y_nbr, x_nbr):
            pl.semaphore_signal(
                barrier_sem, inc=1, device_id=nbr,
                device_id_type=pl.DeviceIdType.MESH,
            )
        pl.semaphore_wait(barrier_sem, 2)

        p1 = []
        for c in range(C):
            rdma = pltpu.make_async_remote_copy(
                src_ref=x_ref.at[pl.ds(my_off + c * h, h), :],
                dst_ref=p1_buf.at[c],
                send_sem=p1_send.at[c],
                recv_sem=p1_recv.at[c],
                device_id=y_nbr,
                device_id_type=pl.DeviceIdType.MESH,
            )
            rdma.start()
            p1.append(rdma)

        p2 = []
        for c in range(C):
            p1[c].wait_recv()
            rows = pl.ds(my_off + c * h, h)
            out_ref[rows, :] = x_ref[rows, :] + p1_buf[c, :, :]
            rdma = pltpu.make_async_remote_copy(
                src_ref=out_ref.at[rows, :],
                dst_ref=out_ref.at[rows, :],
                send_sem=p2_send.at[c],
                recv_sem=p2_recv.at[c],
                device_id=x_nbr,
                device_id_type=pl.DeviceIdType.MESH,
            )
            rdma.start()
            p2.append(rdma)

        for c in range(C):
            recv = pltpu.make_async_remote_copy(
                src_ref=out_ref.at[pl.ds(other_off + c * h, h), :],
                dst_ref=out_ref.at[pl.ds(other_off + c * h, h), :],
                send_sem=p2_send.at[c],
                recv_sem=p2_recv.at[c],
                device_id=x_nbr,
                device_id_type=pl.DeviceIdType.MESH,
            )
            recv.wait_recv()
        for c in range(C):
            p1[c].wait_send()
            p2[c].wait_send()

    return pl.pallas_call(
        body,
        out_shape=jax.ShapeDtypeStruct((m, n), x.dtype),
        in_specs=[pl.BlockSpec(memory_space=pltpu.VMEM)],
        out_specs=pl.BlockSpec(memory_space=pltpu.VMEM),
        scratch_shapes=[
            pltpu.VMEM((C, h, n), x.dtype),
            pltpu.SemaphoreType.DMA((C,)),
            pltpu.SemaphoreType.DMA((C,)),
            pltpu.SemaphoreType.DMA((C,)),
            pltpu.SemaphoreType.DMA((C,)),
        ],
        compiler_params=pltpu.CompilerParams(collective_id=0),
    )(x)


# device time: 35548 ns/iter; 1.4884x vs baseline; 1.0024x over previous
import jax
import jax.numpy as jnp
from jax import lax
from jax.experimental import pallas as pl
from jax.experimental.pallas import tpu as pltpu

K = 8
NCH = 2 * K


def kernel(x):
    m, n = x.shape
    q = m // 4
    h = q // K

    def body(x_ref, out_ref, p1_buf, p1_send, p1_recv, p2_send, p2_recv):
        my_x = lax.axis_index("x")
        my_y = lax.axis_index("y")
        my_z = lax.axis_index("z")
        p = my_z % 2
        y_nbr = (my_x, 1 - my_y, my_z)
        x_nbr = (1 - my_x, my_y, my_z)
        z_par = (my_x, my_y, my_z + 1 - 2 * p)

        c03 = my_x == p
        l_own = jnp.where(c03, 0, 1) * q
        h_own = jnp.where(c03, 3, 2) * q
        l_lack = jnp.where(c03, 1, 0) * q
        h_lack = jnp.where(c03, 2, 3) * q

        barrier_sem = pltpu.get_barrier_semaphore()
        for nbr in (y_nbr, x_nbr, z_par):
            pl.semaphore_signal(
                barrier_sem, inc=1, device_id=nbr,
                device_id_type=pl.DeviceIdType.MESH,
            )
        pl.semaphore_wait(barrier_sem, 3)

        def own_rows(i):
            base = l_own if i % 2 == 0 else h_own
            return pl.ds(base + (i // 2) * h, h)

        p1 = []
        for i in range(NCH):
            rdma = pltpu.make_async_remote_copy(
                src_ref=x_ref.at[own_rows(i), :],
                dst_ref=p1_buf.at[i],
                send_sem=p1_send.at[i],
                recv_sem=p1_recv.at[i],
                device_id=y_nbr,
                device_id_type=pl.DeviceIdType.MESH,
            )
            rdma.start()
            p1.append(rdma)

        p2 = []
        for i in range(NCH):
            p1[i].wait_recv()
            rows = own_rows(i)
            out_ref[rows, :] = x_ref[rows, :] + p1_buf[i, :, :]
            rdma = pltpu.make_async_remote_copy(
                src_ref=out_ref.at[rows, :],
                dst_ref=out_ref.at[rows, :],
                send_sem=p2_send.at[i],
                recv_sem=p2_recv.at[i],
                device_id=x_nbr if i % 2 == 0 else z_par,
                device_id_type=pl.DeviceIdType.MESH,
            )
            rdma.start()
            p2.append(rdma)

        for i in range(NCH):
            base = l_lack if i % 2 == 0 else h_lack
            rows = pl.ds(base + (i // 2) * h, h)
            recv = pltpu.make_async_remote_copy(
                src_ref=out_ref.at[rows, :],
                dst_ref=out_ref.at[rows, :],
                send_sem=p2_send.at[i],
                recv_sem=p2_recv.at[i],
                device_id=x_nbr if i % 2 == 0 else z_par,
                device_id_type=pl.DeviceIdType.MESH,
            )
            recv.wait_recv()
        for i in range(NCH):
            p1[i].wait_send()
            p2[i].wait_send()

    return pl.pallas_call(
        body,
        out_shape=jax.ShapeDtypeStruct((m, n), x.dtype),
        in_specs=[pl.BlockSpec(memory_space=pltpu.VMEM)],
        out_specs=pl.BlockSpec(memory_space=pltpu.VMEM),
        scratch_shapes=[
            pltpu.VMEM((NCH, h, n), x.dtype),
            pltpu.SemaphoreType.DMA((NCH,)),
            pltpu.SemaphoreType.DMA((NCH,)),
            pltpu.SemaphoreType.DMA((NCH,)),
            pltpu.SemaphoreType.DMA((NCH,)),
        ],
        compiler_params=pltpu.CompilerParams(collective_id=0),
    )(x)


# device time: 29855 ns/iter; 1.7722x vs baseline; 1.1907x over previous
import jax
import jax.numpy as jnp
from jax import lax
from jax.experimental import pallas as pl
from jax.experimental.pallas import tpu as pltpu

P = 8
U = 4


def kernel(x):
    m, n = x.shape
    q = m // P
    h = q // U

    def body(x_ref, out_ref, raw_buf, send_sems, recv_sems):
        my_x = lax.axis_index("x")
        my_y = lax.axis_index("y")
        my_z = lax.axis_index("z")
        zp = my_z % 2
        n_x = (1 - my_x, my_y, my_z)
        n_y = (my_x, 1 - my_y, my_z)
        n_z = (my_x, my_y, my_z + 1 - 2 * zp)
        me = 4 * my_x + 2 * my_y + zp

        def rows(part, u):
            return pl.ds(part * q + u * h, h)

        barrier_sem = pltpu.get_barrier_semaphore()
        for nbr in (n_x, n_y, n_z):
            pl.semaphore_signal(
                barrier_sem, inc=1, device_id=nbr,
                device_id_type=pl.DeviceIdType.MESH,
            )
        pl.semaphore_wait(barrier_sem, 3)

        sends = []

        def send(dst_dev, src, dst, recv_slot):
            rdma = pltpu.make_async_remote_copy(
                src_ref=src, dst_ref=dst,
                send_sem=send_sems.at[len(sends)],
                recv_sem=recv_sems.at[recv_slot],
                device_id=dst_dev,
                device_id_type=pl.DeviceIdType.MESH,
            )
            rdma.start()
            sends.append(rdma)

        def recv_wait(dst, recv_slot):
            rdma = pltpu.make_async_remote_copy(
                src_ref=dst, dst_ref=dst,
                send_sem=send_sems.at[0],
                recv_sem=recv_sems.at[recv_slot],
                device_id=n_x,
                device_id_type=pl.DeviceIdType.MESH,
            )
            rdma.wait_recv()

        for u in range(U):
            send(n_y, x_ref.at[rows(me ^ 2, u), :], raw_buf.at[u], u)

        for u in range(U):
            recv_wait(raw_buf.at[u], u)
            r = rows(me, u)
            out_ref[r, :] = x_ref[r, :] + raw_buf[u, :, :]
            send(n_x, out_ref.at[r, :], out_ref.at[r, :], 4 + u)
            send(n_y, out_ref.at[r, :], out_ref.at[r, :], 8 + u)
            send(n_z, out_ref.at[r, :], out_ref.at[r, :], 12 + u)

        for u in range(U):
            rx = rows(me ^ 4, u)
            recv_wait(out_ref.at[rx, :], 4 + u)
            if u <= 2:
                send(n_y, out_ref.at[rx, :], out_ref.at[rx, :], 24 + u)
            ry = rows(me ^ 2, u)
            recv_wait(out_ref.at[ry, :], 8 + u)
            send(n_z, out_ref.at[ry, :], out_ref.at[ry, :], 20 + u)
            if u == 3:
                send(n_x, out_ref.at[ry, :], out_ref.at[ry, :], 27)
            rz = rows(me ^ 1, u)
            recv_wait(out_ref.at[rz, :], 12 + u)
            send(n_x, out_ref.at[rz, :], out_ref.at[rz, :], 16 + u)

        for u in range(U):
            recv_wait(out_ref.at[rows(me ^ 5, u), :], 16 + u)
            rb = rows(me ^ 3, u)
            recv_wait(out_ref.at[rb, :], 20 + u)
            if u <= 1:
                send(n_x, out_ref.at[rb, :], out_ref.at[rb, :], 28 + u)
            rc = rows(me ^ 6, u)
            if u <= 2:
                recv_wait(out_ref.at[rc, :], 24 + u)
                if u == 2:
                    send(n_z, out_ref.at[rc, :], out_ref.at[rc, :], 30)
            else:
                recv_wait(out_ref.at[rc, :], 27)
                send(n_z, out_ref.at[rc, :], out_ref.at[rc, :], 31)

        for u in range(U):
            recv_wait(out_ref.at[rows(me ^ 7, u), :], 28 + u)

        for s in sends:
            s.wait_send()

    return pl.pallas_call(
        body,
        out_shape=jax.ShapeDtypeStruct((m, n), x.dtype),
        in_specs=[pl.BlockSpec(memory_space=pltpu.VMEM)],
        out_specs=pl.BlockSpec(memory_space=pltpu.VMEM),
        scratch_shapes=[
            pltpu.VMEM((U, h, n), x.dtype),
            pltpu.SemaphoreType.DMA((32,)),
            pltpu.SemaphoreType.DMA((32,)),
        ],
        compiler_params=pltpu.CompilerParams(collective_id=0),
    )(x)
